# baseline (speedup 1.0000x reference)
import functools

import jax
import jax.numpy as jnp
from jax import lax
from jax.experimental import pallas as pl
from jax.experimental.pallas import tpu as pltpu
from jax.experimental.pallas import tpu_sc as plsc

_N = 50000
_C = 32
_E = 1600000
_NT = 13
_L = 8
_D = 16

_CHUNK = 128
_TILES = 16
_K = 3
_G = 262
_PER_TILE = _K * _G
_EPC = _PER_TILE * _TILES
_EP = _EPC * _CHUNK
_NP = 50176
_ZCH = 24
_ZTAIL = _NP // _TILES - _ZCH * _CHUNK
_OUT_ROWS = _N // _TILES

_NODES_P = 53248
_NCH_PER_TILE = _NODES_P // (2 * _TILES) // _CHUNK

_sc_mesh = plsc.VectorSubcoreMesh(core_axis_name="c", subcore_axis_name="s")


def _seg_body(gidx, sidx, x_hbm, zblk, out_hbm,
              gi_blk, si_blk, rows_v, h_sh, gsem, ssem, isem):
    c = lax.axis_index("c")
    s = lax.axis_index("s")
    pltpu.sync_copy(zblk, rows_v.at[0, 0])
    zbase = s * (_NP // _TILES)
    for k in range(_ZCH):
        pltpu.sync_copy(rows_v.at[0, 0], h_sh.at[pl.ds(zbase + k * _CHUNK, _CHUNK)])
    pltpu.sync_copy(rows_v.at[0, 0].at[pl.ds(0, _ZTAIL)],
                    h_sh.at[pl.ds(zbase + _ZCH * _CHUNK, _ZTAIL)])
    plsc.subcore_barrier()

    tbase = s * _PER_TILE
    pltpu.sync_copy(gidx.at[c, pl.ds(tbase, _K)], gi_blk.at[0])
    pltpu.sync_copy(sidx.at[c, pl.ds(tbase, _K)], si_blk.at[0])
    for k in range(_K):
        pltpu.async_copy(x_hbm.at[gi_blk.at[0, k]], rows_v.at[0, k], gsem)

    def body(g, carry):
        cur = lax.rem(g, 2)
        nxt = 1 - cur
        for k in range(_K):
            pltpu.make_async_copy(zblk, rows_v.at[cur, k], gsem).wait()
        @pl.when(g > 0)
        def _():
            for k in range(_K):
                pltpu.make_async_copy(zblk, rows_v.at[nxt, k], ssem).wait()
        nb = tbase + lax.min(g + 1, _G - 1) * _K
        gicp = pltpu.async_copy(gidx.at[c, pl.ds(nb, _K)], gi_blk.at[nxt], isem)
        sicp = pltpu.async_copy(sidx.at[c, pl.ds(nb, _K)], si_blk.at[nxt], isem)
        for k in range(_K):
            pltpu.async_copy(rows_v.at[cur, k], h_sh.at[si_blk.at[cur, k]],
                             ssem, add=True)
        gicp.wait()
        sicp.wait()
        for k in range(_K):
            pltpu.async_copy(x_hbm.at[gi_blk.at[nxt, k]], rows_v.at[nxt, k], gsem)
        return carry

    lax.fori_loop(0, _G, body, 0)
    last = lax.rem(_G, 2)
    for k in range(_K):
        pltpu.make_async_copy(zblk, rows_v.at[last, k], gsem).wait()
    for k in range(_K):
        pltpu.make_async_copy(zblk, rows_v.at[1 - last, k], ssem).wait()
    plsc.subcore_barrier()
    obase = s * _OUT_ROWS
    pltpu.sync_copy(h_sh.at[pl.ds(obase, _OUT_ROWS)],
                    out_hbm.at[c].at[pl.ds(obase, _OUT_ROWS)])


_seg_sum = pl.kernel(
    _seg_body,
    out_type=jax.ShapeDtypeStruct((2, _N, _C), jnp.float32),
    mesh=_sc_mesh,
    compiler_params=pltpu.CompilerParams(use_tc_tiling_on_sc=False),
    scratch_types=[
        pltpu.VMEM((2, _K, _CHUNK), jnp.int32),
        pltpu.VMEM((2, _K, _CHUNK), jnp.int32),
        pltpu.VMEM((2, _K, _CHUNK, _C), jnp.float32),
        pltpu.VMEM_SHARED((_NP, _C), jnp.float32),
        pltpu.SemaphoreType.DMA,
        pltpu.SemaphoreType.DMA,
        pltpu.SemaphoreType.DMA,
    ],
)


_NPZ = 51200
_ZSL = _NPZ // _TILES


_CR = _NPZ // 128


def _cnt_body(sidx, izblk, out_hbm, si_blk, cnt_v, isem):
    c = lax.axis_index("c")
    s = lax.axis_index("s")
    for k in range(_TILES):
        pltpu.sync_copy(izblk, cnt_v.at[pl.ds(k * (_CR // _TILES), _CR // _TILES)])

    ones16 = jnp.full((16,), 1.0, jnp.float32)
    tbase = s * _PER_TILE
    pltpu.sync_copy(sidx.at[c, pl.ds(tbase, _K)], si_blk.at[0])

    def body(g, carry):
        cur = lax.rem(g, 2)
        nxt = 1 - cur
        nb = tbase + lax.min(g + 1, _G - 1) * _K
        sicp = pltpu.async_copy(sidx.at[c, pl.ds(nb, _K)], si_blk.at[nxt], isem)
        for k in range(_K):
            for l in range(_CHUNK // 16):
                idx = si_blk[cur, k, pl.ds(l * 16, 16)]
                row = lax.shift_right_logical(idx, 7)
                col = lax.bitwise_and(idx, 127)
                plsc.addupdate_scatter(cnt_v, [row, col], ones16)
        sicp.wait()
        return carry

    lax.fori_loop(0, _G, body, 0)
    pltpu.sync_copy(cnt_v, out_hbm.at[c].at[s])


_seg_cnt = pl.kernel(
    _cnt_body,
    out_type=jax.ShapeDtypeStruct((2, _TILES, _CR, 128), jnp.float32),
    mesh=_sc_mesh,
    compiler_params=pltpu.CompilerParams(use_tc_tiling_on_sc=False,
                                         needs_layout_passes=False),
    scratch_types=[
        pltpu.VMEM((2, _K, _CHUNK), jnp.int32),
        pltpu.VMEM((_CR, 128), jnp.float32),
        pltpu.SemaphoreType.DMA,
    ],
)


def _cntred_body(cnt_ref, inv_ref):
    tot = jnp.sum(cnt_ref[...], axis=1)
    inv_ref[...] = 1.0 / jnp.maximum(tot, 1.0)


def _embed_body(nidx, embed_hbm, out_hbm, ni_v, rows_v, sem):
    c = lax.axis_index("c")
    s = lax.axis_index("s")
    w = c * _TILES + s

    def body(k, carry):
        g = w * _NCH_PER_TILE + k
        pltpu.sync_copy(nidx.at[g], ni_v)
        pltpu.async_copy(embed_hbm.at[ni_v], rows_v, sem).wait()
        pltpu.sync_copy(rows_v, out_hbm.at[pl.ds(g * _CHUNK, _CHUNK)])
        return carry

    lax.fori_loop(0, _NCH_PER_TILE, body, 0)


_embed_gather = pl.kernel(
    _embed_body,
    out_type=jax.ShapeDtypeStruct((_NODES_P, _C), jnp.float32),
    mesh=_sc_mesh,
    compiler_params=pltpu.CompilerParams(use_tc_tiling_on_sc=False),
    scratch_types=[
        pltpu.VMEM((_CHUNK,), jnp.int32),
        pltpu.VMEM((_CHUNK, _C), jnp.float32),
        pltpu.SemaphoreType.DMA,
    ],
)


def _dir_body(hs_ref, inv4_ref, bm_ref, f_ref, g_ref, b_ref, wb_ref, bias_ref, o_ref):
    inv = jnp.dot(inv4_ref[...], bm_ref[...], preferred_element_type=jnp.float32)
    h = hs_ref[...] * inv
    f = f_ref[...]
    m = jnp.mean(h, axis=0) @ f
    hc = h - m[None, :]
    v = jnp.mean(hc * hc, axis=0) @ f
    scale = g_ref[...] * lax.rsqrt(v + 1e-5)
    hn = hc * scale[None, :] + b_ref[...][None, :]
    o = jnp.dot(hn, wb_ref[...], preferred_element_type=jnp.float32)
    o_ref[...] = jnp.maximum(o + bias_ref[...][None, :], 0.0)


def _comb_body(x_ref, a_ref, b_ref, xo_ref):
    xo_ref[...] = x_ref[...] + (a_ref[...] + b_ref[...])


def _final_body(x_ref, ff_ref, mw_ref, mb_ref, vw_ref, vb_ref, mean_ref, var_ref):
    xm = jnp.mean(x_ref[...], axis=0) @ ff_ref[...]
    mean_ref[...] = xm @ mw_ref[...].T + mb_ref[...]
    var_ref[...] = xm @ vw_ref[...].T + vb_ref[...]


_f32 = jnp.float32


def kernel(nodes, sources, targets, embed, bn_gamma, bn_beta, conv_W, conv_b,
           mean_W, mean_b, var_W, var_b):
    src = sources.astype(jnp.int32)
    tgt = targets.astype(jnp.int32)
    pad_g = jnp.zeros((_EP - _E,), jnp.int32)
    pad_s = jnp.full((_EP - _E,), _N, jnp.int32)
    g0 = jnp.concatenate([src, pad_g]).reshape(_EPC, _CHUNK)
    g1 = jnp.concatenate([tgt, pad_g]).reshape(_EPC, _CHUNK)
    s0 = jnp.concatenate([tgt, pad_s]).reshape(_EPC, _CHUNK)
    s1 = jnp.concatenate([src, pad_s]).reshape(_EPC, _CHUNK)
    gidx = jnp.stack([g0, g1])
    sidx = jnp.stack([s0, s1])
    zblk = jnp.zeros((_CHUNK, _C), _f32)
    nidx = jnp.concatenate(
        [nodes.astype(jnp.int32), jnp.zeros((_NODES_P - _N,), jnp.int32)]
    ).reshape(_NODES_P // _CHUNK, _CHUNK)

    npk = _N // 4
    fold = jnp.kron(jnp.ones((4, 4), _f32) / 4.0, jnp.eye(_C, dtype=_f32))
    foldf = jnp.kron(jnp.ones((4, 1), _f32) / 4.0, jnp.eye(_C, dtype=_f32))
    wb = jnp.kron(jnp.eye(4, dtype=_f32),
                  conv_W.transpose(0, 1, 3, 2))
    g4 = jnp.tile(bn_gamma, (1, 1, 4))
    b4 = jnp.tile(bn_beta, (1, 1, 4))
    bias4 = jnp.tile(conv_b, (1, 1, 4))

    izblk = jnp.zeros((_CR // _TILES, 128), _f32)
    cntp = _seg_cnt(sidx, izblk)
    invg = pl.pallas_call(
        _cntred_body, out_shape=jax.ShapeDtypeStruct((2, _CR, 128), _f32),
    )(cntp)
    inv4 = invg.reshape(2, _NPZ)[:, :_N].reshape(2, npk, 4)
    bmat = jnp.kron(jnp.eye(4, dtype=_f32), jnp.ones((1, _C), _f32))
    x = _embed_gather(nidx, embed)[:_N].reshape(npk, 128)

    dir_call = pl.pallas_call(
        _dir_body, out_shape=jax.ShapeDtypeStruct((npk, 128), _f32),
    )
    comb_call = pl.pallas_call(
        _comb_body, out_shape=jax.ShapeDtypeStruct((npk, 128), _f32),
    )
    for i in range(_L):
        hs2 = _seg_sum(gidx, sidx, x.reshape(_N, _C), zblk).reshape(2, npk, 128)
        o0 = dir_call(hs2[0], inv4[0], bmat, fold, g4[i, 0], b4[i, 0],
                      wb[i, 0], bias4[i, 0])
        o1 = dir_call(hs2[1], inv4[1], bmat, fold, g4[i, 1], b4[i, 1],
                      wb[i, 1], bias4[i, 1])
        x = comb_call(x, o0, o1)

    mean, var = pl.pallas_call(
        _final_body,
        out_shape=(jax.ShapeDtypeStruct((_D,), _f32),
                   jax.ShapeDtypeStruct((_D,), _f32)),
    )(x, foldf, mean_W, mean_b, var_W, var_b)
    return (mean, var)


# speedup vs baseline: 1.1375x; 1.1375x over previous
import functools

import jax
import jax.numpy as jnp
from jax import lax
from jax.experimental import pallas as pl
from jax.experimental.pallas import tpu as pltpu
from jax.experimental.pallas import tpu_sc as plsc

_N = 50000
_C = 32
_E = 1600000
_NT = 13
_L = 8
_D = 16

_CHUNK = 128
_TILES = 16
_K = 3
_G = 262
_PER_TILE = _K * _G
_EPC = _PER_TILE * _TILES
_EP = _EPC * _CHUNK
_NP = 50176
_ZCH = 24
_ZTAIL = _NP // _TILES - _ZCH * _CHUNK
_OUT_ROWS = _N // _TILES

_NODES_P = 53248
_NCH_PER_TILE = _NODES_P // (2 * _TILES) // _CHUNK

_sc_mesh = plsc.VectorSubcoreMesh(core_axis_name="c", subcore_axis_name="s")


def _seg_body(gidx, sidx, x_hbm, zblk, out_hbm,
              gi_blk, si_blk, rows_v, h_sh, gsem, ssem, isem):
    c = lax.axis_index("c")
    s = lax.axis_index("s")
    pltpu.sync_copy(zblk, rows_v.at[0, 0])
    zbase = s * (_NP // _TILES)
    for k in range(_ZCH):
        pltpu.sync_copy(rows_v.at[0, 0], h_sh.at[pl.ds(zbase + k * _CHUNK, _CHUNK)])
    pltpu.sync_copy(rows_v.at[0, 0].at[pl.ds(0, _ZTAIL)],
                    h_sh.at[pl.ds(zbase + _ZCH * _CHUNK, _ZTAIL)])
    plsc.subcore_barrier()

    tbase = s * _PER_TILE
    pltpu.sync_copy(gidx.at[c, pl.ds(tbase, _K)], gi_blk.at[0])
    pltpu.sync_copy(sidx.at[c, pl.ds(tbase, _K)], si_blk.at[0])
    for k in range(_K):
        pltpu.async_copy(x_hbm.at[gi_blk.at[0, k]], rows_v.at[0, k], gsem)

    def body(g, carry):
        cur = lax.rem(g, 2)
        nxt = 1 - cur
        for k in range(_K):
            pltpu.make_async_copy(zblk, rows_v.at[cur, k], gsem).wait()
        @pl.when(g > 0)
        def _():
            for k in range(_K):
                pltpu.make_async_copy(zblk, rows_v.at[nxt, k], ssem).wait()
        nb = tbase + lax.min(g + 1, _G - 1) * _K
        gicp = pltpu.async_copy(gidx.at[c, pl.ds(nb, _K)], gi_blk.at[nxt], isem)
        sicp = pltpu.async_copy(sidx.at[c, pl.ds(nb, _K)], si_blk.at[nxt], isem)
        for k in range(_K):
            pltpu.async_copy(rows_v.at[cur, k], h_sh.at[si_blk.at[cur, k]],
                             ssem, add=True)
        gicp.wait()
        sicp.wait()
        for k in range(_K):
            pltpu.async_copy(x_hbm.at[gi_blk.at[nxt, k]], rows_v.at[nxt, k], gsem)
        return carry

    lax.fori_loop(0, _G, body, 0)
    last = lax.rem(_G, 2)
    for k in range(_K):
        pltpu.make_async_copy(zblk, rows_v.at[last, k], gsem).wait()
    for k in range(_K):
        pltpu.make_async_copy(zblk, rows_v.at[1 - last, k], ssem).wait()
    plsc.subcore_barrier()
    obase = s * _OUT_ROWS
    pltpu.sync_copy(h_sh.at[pl.ds(obase, _OUT_ROWS)],
                    out_hbm.at[c].at[pl.ds(obase, _OUT_ROWS)])


_seg_sum = pl.kernel(
    _seg_body,
    out_type=jax.ShapeDtypeStruct((2, _N, _C), jnp.float32),
    mesh=_sc_mesh,
    compiler_params=pltpu.CompilerParams(use_tc_tiling_on_sc=False),
    scratch_types=[
        pltpu.VMEM((2, _K, _CHUNK), jnp.int32),
        pltpu.VMEM((2, _K, _CHUNK), jnp.int32),
        pltpu.VMEM((2, _K, _CHUNK, _C), jnp.float32),
        pltpu.VMEM_SHARED((_NP, _C), jnp.float32),
        pltpu.SemaphoreType.DMA,
        pltpu.SemaphoreType.DMA,
        pltpu.SemaphoreType.DMA,
    ],
)


_NPZ = 51200
_ZSL = _NPZ // _TILES


_CR = _NPZ // 128


def _cnt_body(sidx, izblk, out_hbm, si_blk, cnt_v, isem):
    c = lax.axis_index("c")
    s = lax.axis_index("s")
    for k in range(_TILES):
        pltpu.sync_copy(izblk, cnt_v.at[pl.ds(k * (_CR // _TILES), _CR // _TILES)])

    ones16 = jnp.full((16,), 1.0, jnp.float32)
    tbase = s * _PER_TILE
    pltpu.sync_copy(sidx.at[c, pl.ds(tbase, _K)], si_blk.at[0])

    def body(g, carry):
        cur = lax.rem(g, 2)
        nxt = 1 - cur
        nb = tbase + lax.min(g + 1, _G - 1) * _K
        sicp = pltpu.async_copy(sidx.at[c, pl.ds(nb, _K)], si_blk.at[nxt], isem)
        for k in range(_K):
            for l in range(_CHUNK // 16):
                idx = si_blk[cur, k, pl.ds(l * 16, 16)]
                row = lax.shift_right_logical(idx, 7)
                col = lax.bitwise_and(idx, 127)
                plsc.addupdate_scatter(cnt_v, [row, col], ones16)
        sicp.wait()
        return carry

    lax.fori_loop(0, _G, body, 0)
    pltpu.sync_copy(cnt_v, out_hbm.at[c].at[s])


_seg_cnt = pl.kernel(
    _cnt_body,
    out_type=jax.ShapeDtypeStruct((2, _TILES, _CR, 128), jnp.float32),
    mesh=_sc_mesh,
    compiler_params=pltpu.CompilerParams(use_tc_tiling_on_sc=False,
                                         needs_layout_passes=False),
    scratch_types=[
        pltpu.VMEM((2, _K, _CHUNK), jnp.int32),
        pltpu.VMEM((_CR, 128), jnp.float32),
        pltpu.SemaphoreType.DMA,
    ],
)


def _cntred_body(cnt_ref, inv_ref):
    tot = jnp.sum(cnt_ref[...], axis=1)
    inv_ref[...] = 1.0 / jnp.maximum(tot, 1.0)


def _embed_body(nidx, embed_hbm, out_hbm, ni_v, rows_v, sem):
    c = lax.axis_index("c")
    s = lax.axis_index("s")
    w = c * _TILES + s

    def body(k, carry):
        g = w * _NCH_PER_TILE + k
        pltpu.sync_copy(nidx.at[g], ni_v)
        pltpu.async_copy(embed_hbm.at[ni_v], rows_v, sem).wait()
        pltpu.sync_copy(rows_v, out_hbm.at[pl.ds(g * _CHUNK, _CHUNK)])
        return carry

    lax.fori_loop(0, _NCH_PER_TILE, body, 0)


_embed_gather = pl.kernel(
    _embed_body,
    out_type=jax.ShapeDtypeStruct((_NODES_P, _C), jnp.float32),
    mesh=_sc_mesh,
    compiler_params=pltpu.CompilerParams(use_tc_tiling_on_sc=False),
    scratch_types=[
        pltpu.VMEM((_CHUNK,), jnp.int32),
        pltpu.VMEM((_CHUNK, _C), jnp.float32),
        pltpu.SemaphoreType.DMA,
    ],
)


def _invpack_body(i4_ref, bm_ref, o_ref):
    for d in range(2):
        o_ref[d] = jnp.dot(i4_ref[d], bm_ref[...],
                           preferred_element_type=jnp.float32)


def _layer_body(x_ref, hs_ref, inv_ref, f_ref, g_ref, b_ref, wb_ref, bias_ref,
                xo_ref):
    f = f_ref[...]
    acc = x_ref[...]
    for d in range(2):
        h = hs_ref[d] * inv_ref[d]
        m = jnp.mean(h, axis=0) @ f
        hc = h - m[None, :]
        v = jnp.mean(hc * hc, axis=0) @ f
        scale = g_ref[d] * lax.rsqrt(v + 1e-5)
        hn = hc * scale[None, :] + b_ref[d][None, :]
        o = jnp.dot(hn, wb_ref[d], preferred_element_type=jnp.float32)
        acc = acc + jnp.maximum(o + bias_ref[d][None, :], 0.0)
    xo_ref[...] = acc


def _final_body(x_ref, ff_ref, mw_ref, mb_ref, vw_ref, vb_ref, mean_ref, var_ref):
    xm = jnp.mean(x_ref[...], axis=0) @ ff_ref[...]
    mean_ref[...] = xm @ mw_ref[...].T + mb_ref[...]
    var_ref[...] = xm @ vw_ref[...].T + vb_ref[...]


_f32 = jnp.float32


def kernel(nodes, sources, targets, embed, bn_gamma, bn_beta, conv_W, conv_b,
           mean_W, mean_b, var_W, var_b):
    src = sources.astype(jnp.int32)
    tgt = targets.astype(jnp.int32)
    pad_g = jnp.zeros((_EP - _E,), jnp.int32)
    pad_s = jnp.full((_EP - _E,), _N, jnp.int32)
    g0 = jnp.concatenate([src, pad_g]).reshape(_EPC, _CHUNK)
    g1 = jnp.concatenate([tgt, pad_g]).reshape(_EPC, _CHUNK)
    s0 = jnp.concatenate([tgt, pad_s]).reshape(_EPC, _CHUNK)
    s1 = jnp.concatenate([src, pad_s]).reshape(_EPC, _CHUNK)
    gidx = jnp.stack([g0, g1])
    sidx = jnp.stack([s0, s1])
    zblk = jnp.zeros((_CHUNK, _C), _f32)
    nidx = jnp.concatenate(
        [nodes.astype(jnp.int32), jnp.zeros((_NODES_P - _N,), jnp.int32)]
    ).reshape(_NODES_P // _CHUNK, _CHUNK)

    npk = _N // 4
    fold = jnp.kron(jnp.ones((4, 4), _f32) / 4.0, jnp.eye(_C, dtype=_f32))
    foldf = jnp.kron(jnp.ones((4, 1), _f32) / 4.0, jnp.eye(_C, dtype=_f32))
    wb = jnp.kron(jnp.eye(4, dtype=_f32),
                  conv_W.transpose(0, 1, 3, 2))
    g4 = jnp.tile(bn_gamma, (1, 1, 4))
    b4 = jnp.tile(bn_beta, (1, 1, 4))
    bias4 = jnp.tile(conv_b, (1, 1, 4))

    izblk = jnp.zeros((_CR // _TILES, 128), _f32)
    cntp = _seg_cnt(sidx, izblk)
    invg = pl.pallas_call(
        _cntred_body, out_shape=jax.ShapeDtypeStruct((2, _CR, 128), _f32),
    )(cntp)
    inv4 = invg.reshape(2, _NPZ)[:, :_N].reshape(2, npk, 4)
    bmat = jnp.kron(jnp.eye(4, dtype=_f32), jnp.ones((1, _C), _f32))
    inv2 = pl.pallas_call(
        _invpack_body, out_shape=jax.ShapeDtypeStruct((2, npk, 128), _f32),
    )(inv4, bmat)
    x = _embed_gather(nidx, embed)[:_N].reshape(npk, 128)

    layer_call = pl.pallas_call(
        _layer_body, out_shape=jax.ShapeDtypeStruct((npk, 128), _f32),
    )
    for i in range(_L):
        hs2 = _seg_sum(gidx, sidx, x.reshape(_N, _C), zblk).reshape(2, npk, 128)
        x = layer_call(x, hs2, inv2, fold, g4[i], b4[i], wb[i], bias4[i])

    mean, var = pl.pallas_call(
        _final_body,
        out_shape=(jax.ShapeDtypeStruct((_D,), _f32),
                   jax.ShapeDtypeStruct((_D,), _f32)),
    )(x, foldf, mean_W, mean_b, var_W, var_b)
    return (mean, var)


# speedup vs baseline: 1.1379x; 1.0003x over previous
import functools

import jax
import jax.numpy as jnp
from jax import lax
from jax.experimental import pallas as pl
from jax.experimental.pallas import tpu as pltpu
from jax.experimental.pallas import tpu_sc as plsc

_N = 50000
_C = 32
_E = 1600000
_NT = 13
_L = 8
_D = 16

_CHUNK = 128
_TILES = 16
_K = 3
_G = 262
_PER_TILE = _K * _G
_EPC = _PER_TILE * _TILES
_EP = _EPC * _CHUNK
_NP = 50176
_ZCH = 24
_ZTAIL = _NP // _TILES - _ZCH * _CHUNK
_OUT_ROWS = _N // _TILES

_NODES_P = 53248
_NCH_PER_TILE = _NODES_P // (2 * _TILES) // _CHUNK

_sc_mesh = plsc.VectorSubcoreMesh(core_axis_name="c", subcore_axis_name="s")


def _seg_body(gidx, sidx, x_hbm, zblk, out_hbm,
              gi_blk, si_blk, rows_v, h_sh, gsem, ssem, isem):
    c = lax.axis_index("c")
    s = lax.axis_index("s")
    pltpu.sync_copy(zblk, rows_v.at[0, 0])
    zbase = s * (_NP // _TILES)
    for k in range(_ZCH):
        pltpu.sync_copy(rows_v.at[0, 0], h_sh.at[pl.ds(zbase + k * _CHUNK, _CHUNK)])
    pltpu.sync_copy(rows_v.at[0, 0].at[pl.ds(0, _ZTAIL)],
                    h_sh.at[pl.ds(zbase + _ZCH * _CHUNK, _ZTAIL)])
    plsc.subcore_barrier()

    tbase = s * _PER_TILE
    pltpu.sync_copy(gidx.at[c, pl.ds(tbase, _K)], gi_blk.at[0])
    pltpu.sync_copy(sidx.at[c, pl.ds(tbase, _K)], si_blk.at[0])
    for k in range(_K):
        pltpu.async_copy(x_hbm.at[gi_blk.at[0, k]], rows_v.at[0, k], gsem)

    def body(g, carry):
        cur = lax.rem(g, 2)
        nxt = 1 - cur
        for k in range(_K):
            pltpu.make_async_copy(zblk, rows_v.at[cur, k], gsem).wait()
        @pl.when(g > 0)
        def _():
            for k in range(_K):
                pltpu.make_async_copy(zblk, rows_v.at[nxt, k], ssem).wait()
        nb = tbase + lax.min(g + 1, _G - 1) * _K
        gicp = pltpu.async_copy(gidx.at[c, pl.ds(nb, _K)], gi_blk.at[nxt], isem)
        sicp = pltpu.async_copy(sidx.at[c, pl.ds(nb, _K)], si_blk.at[nxt], isem)
        for k in range(_K):
            pltpu.async_copy(rows_v.at[cur, k], h_sh.at[si_blk.at[cur, k]],
                             ssem, add=True)
        gicp.wait()
        sicp.wait()
        for k in range(_K):
            pltpu.async_copy(x_hbm.at[gi_blk.at[nxt, k]], rows_v.at[nxt, k], gsem)
        return carry

    lax.fori_loop(0, _G, body, 0)
    last = lax.rem(_G, 2)
    for k in range(_K):
        pltpu.make_async_copy(zblk, rows_v.at[last, k], gsem).wait()
    for k in range(_K):
        pltpu.make_async_copy(zblk, rows_v.at[1 - last, k], ssem).wait()
    plsc.subcore_barrier()
    obase = s * _OUT_ROWS
    pltpu.sync_copy(h_sh.at[pl.ds(obase, _OUT_ROWS)],
                    out_hbm.at[c].at[pl.ds(obase, _OUT_ROWS)])


_seg_sum = pl.kernel(
    _seg_body,
    out_type=jax.ShapeDtypeStruct((2, _N, _C), jnp.float32),
    mesh=_sc_mesh,
    compiler_params=pltpu.CompilerParams(use_tc_tiling_on_sc=False),
    scratch_types=[
        pltpu.VMEM((2, _K, _CHUNK), jnp.int32),
        pltpu.VMEM((2, _K, _CHUNK), jnp.int32),
        pltpu.VMEM((2, _K, _CHUNK, _C), jnp.float32),
        pltpu.VMEM_SHARED((_NP, _C), jnp.float32),
        pltpu.SemaphoreType.DMA,
        pltpu.SemaphoreType.DMA,
        pltpu.SemaphoreType.DMA,
    ],
)


_NPZ = 51200
_ZSL = _NPZ // _TILES


_CR = _NPZ // 128


def _cnt_body(sidx, izblk, out_hbm, si_blk, cnt_v, isem):
    c = lax.axis_index("c")
    s = lax.axis_index("s")
    for k in range(_TILES):
        pltpu.sync_copy(izblk, cnt_v.at[pl.ds(k * (_CR // _TILES), _CR // _TILES)])

    ones16 = jnp.full((16,), 1.0, jnp.float32)
    tbase = s * _PER_TILE
    pltpu.sync_copy(sidx.at[c, pl.ds(tbase, _K)], si_blk.at[0])

    def body(g, carry):
        cur = lax.rem(g, 2)
        nxt = 1 - cur
        nb = tbase + lax.min(g + 1, _G - 1) * _K
        sicp = pltpu.async_copy(sidx.at[c, pl.ds(nb, _K)], si_blk.at[nxt], isem)
        for k in range(_K):
            for l in range(_CHUNK // 16):
                idx = si_blk[cur, k, pl.ds(l * 16, 16)]
                row = lax.shift_right_logical(idx, 7)
                col = lax.bitwise_and(idx, 127)
                plsc.addupdate_scatter(cnt_v, [row, col], ones16)
        sicp.wait()
        return carry

    lax.fori_loop(0, _G, body, 0)
    pltpu.sync_copy(cnt_v, out_hbm.at[c].at[s])


_seg_cnt = pl.kernel(
    _cnt_body,
    out_type=jax.ShapeDtypeStruct((2, _TILES, _CR, 128), jnp.float32),
    mesh=_sc_mesh,
    compiler_params=pltpu.CompilerParams(use_tc_tiling_on_sc=False,
                                         needs_layout_passes=False),
    scratch_types=[
        pltpu.VMEM((2, _K, _CHUNK), jnp.int32),
        pltpu.VMEM((_CR, 128), jnp.float32),
        pltpu.SemaphoreType.DMA,
    ],
)


def _cntred_body(cnt_ref, inv_ref):
    tot = jnp.sum(cnt_ref[...], axis=1)
    inv_ref[...] = 1.0 / jnp.maximum(tot, 1.0)


def _embed_body(nidx, embed_hbm, out_hbm, ni_v, rows_v, isem, gsem, osem):
    c = lax.axis_index("c")
    s = lax.axis_index("s")
    w = c * _TILES + s
    base = w * _NCH_PER_TILE
    icp = pltpu.async_copy(nidx.at[pl.ds(base, _NCH_PER_TILE)], ni_v, isem)
    icp.wait()
    for k in range(_NCH_PER_TILE):
        pltpu.async_copy(embed_hbm.at[ni_v.at[k]], rows_v.at[k], gsem)
    for k in range(_NCH_PER_TILE):
        pltpu.make_async_copy(embed_hbm.at[ni_v.at[k]], rows_v.at[k], gsem).wait()
    for k in range(_NCH_PER_TILE):
        pltpu.async_copy(
            rows_v.at[k], out_hbm.at[pl.ds((base + k) * _CHUNK, _CHUNK)], osem)
    for k in range(_NCH_PER_TILE):
        pltpu.make_async_copy(
            rows_v.at[k], out_hbm.at[pl.ds((base + k) * _CHUNK, _CHUNK)],
            osem).wait()


_embed_gather = pl.kernel(
    _embed_body,
    out_type=jax.ShapeDtypeStruct((_NODES_P, _C), jnp.float32),
    mesh=_sc_mesh,
    compiler_params=pltpu.CompilerParams(use_tc_tiling_on_sc=False),
    scratch_types=[
        pltpu.VMEM((_NCH_PER_TILE, _CHUNK), jnp.int32),
        pltpu.VMEM((_NCH_PER_TILE, _CHUNK, _C), jnp.float32),
        pltpu.SemaphoreType.DMA,
        pltpu.SemaphoreType.DMA,
        pltpu.SemaphoreType.DMA,
    ],
)


def _invpack_body(i4_ref, bm_ref, o_ref):
    for d in range(2):
        o_ref[d] = jnp.dot(i4_ref[d], bm_ref[...],
                           preferred_element_type=jnp.float32)


def _layer_body(x_ref, hs_ref, inv_ref, f_ref, g_ref, b_ref, wb_ref, bias_ref,
                xo_ref):
    f = f_ref[...]
    acc = x_ref[...]
    for d in range(2):
        h = hs_ref[d] * inv_ref[d]
        m = jnp.mean(h, axis=0) @ f
        hc = h - m[None, :]
        v = jnp.mean(hc * hc, axis=0) @ f
        scale = g_ref[d] * lax.rsqrt(v + 1e-5)
        hn = hc * scale[None, :] + b_ref[d][None, :]
        o = jnp.dot(hn, wb_ref[d], preferred_element_type=jnp.float32)
        acc = acc + jnp.maximum(o + bias_ref[d][None, :], 0.0)
    xo_ref[...] = acc


def _final_body(x_ref, ff_ref, mw_ref, mb_ref, vw_ref, vb_ref, mean_ref, var_ref):
    xm = jnp.mean(x_ref[...], axis=0) @ ff_ref[...]
    mean_ref[...] = xm @ mw_ref[...].T + mb_ref[...]
    var_ref[...] = xm @ vw_ref[...].T + vb_ref[...]


_f32 = jnp.float32


def kernel(nodes, sources, targets, embed, bn_gamma, bn_beta, conv_W, conv_b,
           mean_W, mean_b, var_W, var_b):
    src = sources.astype(jnp.int32)
    tgt = targets.astype(jnp.int32)
    pad_g = jnp.zeros((_EP - _E,), jnp.int32)
    pad_s = jnp.full((_EP - _E,), _N, jnp.int32)
    g0 = jnp.concatenate([src, pad_g]).reshape(_EPC, _CHUNK)
    g1 = jnp.concatenate([tgt, pad_g]).reshape(_EPC, _CHUNK)
    s0 = jnp.concatenate([tgt, pad_s]).reshape(_EPC, _CHUNK)
    s1 = jnp.concatenate([src, pad_s]).reshape(_EPC, _CHUNK)
    gidx = jnp.stack([g0, g1])
    sidx = jnp.stack([s0, s1])
    zblk = jnp.zeros((_CHUNK, _C), _f32)
    nidx = jnp.concatenate(
        [nodes.astype(jnp.int32), jnp.zeros((_NODES_P - _N,), jnp.int32)]
    ).reshape(_NODES_P // _CHUNK, _CHUNK)

    npk = _N // 4
    fold = jnp.kron(jnp.ones((4, 4), _f32) / 4.0, jnp.eye(_C, dtype=_f32))
    foldf = jnp.kron(jnp.ones((4, 1), _f32) / 4.0, jnp.eye(_C, dtype=_f32))
    wb = jnp.kron(jnp.eye(4, dtype=_f32),
                  conv_W.transpose(0, 1, 3, 2))
    g4 = jnp.tile(bn_gamma, (1, 1, 4))
    b4 = jnp.tile(bn_beta, (1, 1, 4))
    bias4 = jnp.tile(conv_b, (1, 1, 4))

    izblk = jnp.zeros((_CR // _TILES, 128), _f32)
    cntp = _seg_cnt(sidx, izblk)
    invg = pl.pallas_call(
        _cntred_body, out_shape=jax.ShapeDtypeStruct((2, _CR, 128), _f32),
    )(cntp)
    inv4 = invg.reshape(2, _NPZ)[:, :_N].reshape(2, npk, 4)
    bmat = jnp.kron(jnp.eye(4, dtype=_f32), jnp.ones((1, _C), _f32))
    inv2 = pl.pallas_call(
        _invpack_body, out_shape=jax.ShapeDtypeStruct((2, npk, 128), _f32),
    )(inv4, bmat)
    x = _embed_gather(nidx, embed)[:_N].reshape(npk, 128)

    layer_call = pl.pallas_call(
        _layer_body, out_shape=jax.ShapeDtypeStruct((npk, 128), _f32),
    )
    for i in range(_L):
        hs2 = _seg_sum(gidx, sidx, x.reshape(_N, _C), zblk).reshape(2, npk, 128)
        x = layer_call(x, hs2, inv2, fold, g4[i], b4[i], wb[i], bias4[i])

    mean, var = pl.pallas_call(
        _final_body,
        out_shape=(jax.ShapeDtypeStruct((_D,), _f32),
                   jax.ShapeDtypeStruct((_D,), _f32)),
    )(x, foldf, mean_W, mean_b, var_W, var_b)
    return (mean, var)


# speedup vs baseline: 1.1379x; 1.0001x over previous
import jax
import jax.numpy as jnp
from jax import lax
from jax.experimental import pallas as pl
from jax.experimental.pallas import tpu as pltpu
from jax.experimental.pallas import tpu_sc as plsc

_N = 50000
_C = 32
_E = 1600000
_NT = 13
_L = 8
_D = 16

_CHUNK = 128
_TILES = 16
_K = 3
_G = 262
_PER_TILE = _K * _G
_EPC = _PER_TILE * _TILES
_EP = _EPC * _CHUNK
_NP = 50176
_ZCH = 24
_ZTAIL = _NP // _TILES - _ZCH * _CHUNK
_OUT_ROWS = _N // _TILES

_NODES_P = 53248
_NCH_PER_TILE = _NODES_P // (2 * _TILES) // _CHUNK

_sc_mesh = plsc.VectorSubcoreMesh(core_axis_name="c", subcore_axis_name="s")


def _seg_body(gidx, sidx, x_hbm, zblk, out_hbm,
              gi_blk, si_blk, rows_v, h_sh, gsem, ssem, isem):
    c = lax.axis_index("c")
    s = lax.axis_index("s")
    pltpu.sync_copy(zblk, rows_v.at[0, 0])
    zbase = s * (_NP // _TILES)
    for k in range(_ZCH):
        pltpu.sync_copy(rows_v.at[0, 0], h_sh.at[pl.ds(zbase + k * _CHUNK, _CHUNK)])
    pltpu.sync_copy(rows_v.at[0, 0].at[pl.ds(0, _ZTAIL)],
                    h_sh.at[pl.ds(zbase + _ZCH * _CHUNK, _ZTAIL)])
    plsc.subcore_barrier()

    tbase = s * _PER_TILE
    pltpu.sync_copy(gidx.at[c, pl.ds(tbase, _K)], gi_blk.at[0])
    pltpu.sync_copy(sidx.at[c, pl.ds(tbase, _K)], si_blk.at[0])
    for k in range(_K):
        pltpu.async_copy(x_hbm.at[gi_blk.at[0, k]], rows_v.at[0, k], gsem)

    def body(g, carry):
        cur = lax.rem(g, 2)
        nxt = 1 - cur
        for k in range(_K):
            pltpu.make_async_copy(zblk, rows_v.at[cur, k], gsem).wait()
        @pl.when(g > 0)
        def _():
            for k in range(_K):
                pltpu.make_async_copy(zblk, rows_v.at[nxt, k], ssem).wait()
        nb = tbase + lax.min(g + 1, _G - 1) * _K
        gicp = pltpu.async_copy(gidx.at[c, pl.ds(nb, _K)], gi_blk.at[nxt], isem)
        sicp = pltpu.async_copy(sidx.at[c, pl.ds(nb, _K)], si_blk.at[nxt], isem)
        for k in range(_K):
            pltpu.async_copy(rows_v.at[cur, k], h_sh.at[si_blk.at[cur, k]],
                             ssem, add=True)
        gicp.wait()
        sicp.wait()
        for k in range(_K):
            pltpu.async_copy(x_hbm.at[gi_blk.at[nxt, k]], rows_v.at[nxt, k], gsem)
        return carry

    lax.fori_loop(0, _G, body, 0)
    last = lax.rem(_G, 2)
    for k in range(_K):
        pltpu.make_async_copy(zblk, rows_v.at[last, k], gsem).wait()
    for k in range(_K):
        pltpu.make_async_copy(zblk, rows_v.at[1 - last, k], ssem).wait()
    plsc.subcore_barrier()
    obase = s * _OUT_ROWS
    pltpu.sync_copy(h_sh.at[pl.ds(obase, _OUT_ROWS)],
                    out_hbm.at[c].at[pl.ds(obase, _OUT_ROWS)])


_seg_sum = pl.kernel(
    _seg_body,
    out_type=jax.ShapeDtypeStruct((2, _N, _C), jnp.float32),
    mesh=_sc_mesh,
    compiler_params=pltpu.CompilerParams(use_tc_tiling_on_sc=False),
    scratch_types=[
        pltpu.VMEM((2, _K, _CHUNK), jnp.int32),
        pltpu.VMEM((2, _K, _CHUNK), jnp.int32),
        pltpu.VMEM((2, _K, _CHUNK, _C), jnp.float32),
        pltpu.VMEM_SHARED((_NP, _C), jnp.float32),
        pltpu.SemaphoreType.DMA,
        pltpu.SemaphoreType.DMA,
        pltpu.SemaphoreType.DMA,
    ],
)


_NPZ = 51200
_CR = _NPZ // 128


def _cnt_body(sidx, izblk, out_hbm, si_blk, cnt_v, isem):
    c = lax.axis_index("c")
    s = lax.axis_index("s")
    for k in range(_TILES):
        pltpu.sync_copy(izblk, cnt_v.at[pl.ds(k * (_CR // _TILES), _CR // _TILES)])

    ones16 = jnp.full((16,), 1.0, jnp.float32)
    tbase = s * _PER_TILE
    pltpu.sync_copy(sidx.at[c, pl.ds(tbase, _K)], si_blk.at[0])

    def body(g, carry):
        cur = lax.rem(g, 2)
        nxt = 1 - cur
        nb = tbase + lax.min(g + 1, _G - 1) * _K
        sicp = pltpu.async_copy(sidx.at[c, pl.ds(nb, _K)], si_blk.at[nxt], isem)
        for k in range(_K):
            for l in range(_CHUNK // 16):
                idx = si_blk[cur, k, pl.ds(l * 16, 16)]
                row = lax.shift_right_logical(idx, 7)
                col = lax.bitwise_and(idx, 127)
                plsc.addupdate_scatter(cnt_v, [row, col], ones16)
        sicp.wait()
        return carry

    lax.fori_loop(0, _G, body, 0)
    pltpu.sync_copy(cnt_v, out_hbm.at[c].at[s])


_seg_cnt = pl.kernel(
    _cnt_body,
    out_type=jax.ShapeDtypeStruct((2, _TILES, _CR, 128), jnp.float32),
    mesh=_sc_mesh,
    compiler_params=pltpu.CompilerParams(use_tc_tiling_on_sc=False,
                                         needs_layout_passes=False),
    scratch_types=[
        pltpu.VMEM((2, _K, _CHUNK), jnp.int32),
        pltpu.VMEM((_CR, 128), jnp.float32),
        pltpu.SemaphoreType.DMA,
    ],
)


def _cntred_body(cnt_ref, inv_ref):
    tot = jnp.sum(cnt_ref[...], axis=1)
    inv_ref[...] = 1.0 / jnp.maximum(tot, 1.0)


def _embed_body(nidx, embed_hbm, out_hbm, ni_v, rows_v, isem, gsem, osem):
    c = lax.axis_index("c")
    s = lax.axis_index("s")
    w = c * _TILES + s
    base = w * _NCH_PER_TILE
    icp = pltpu.async_copy(nidx.at[pl.ds(base, _NCH_PER_TILE)], ni_v, isem)
    icp.wait()
    for k in range(_NCH_PER_TILE):
        pltpu.async_copy(embed_hbm.at[ni_v.at[k]], rows_v.at[k], gsem)
    for k in range(_NCH_PER_TILE):
        pltpu.make_async_copy(embed_hbm.at[ni_v.at[k]], rows_v.at[k], gsem).wait()
    for k in range(_NCH_PER_TILE):
        pltpu.async_copy(
            rows_v.at[k], out_hbm.at[pl.ds((base + k) * _CHUNK, _CHUNK)], osem)
    for k in range(_NCH_PER_TILE):
        pltpu.make_async_copy(
            rows_v.at[k], out_hbm.at[pl.ds((base + k) * _CHUNK, _CHUNK)],
            osem).wait()


_embed_gather = pl.kernel(
    _embed_body,
    out_type=jax.ShapeDtypeStruct((_NODES_P, _C), jnp.float32),
    mesh=_sc_mesh,
    compiler_params=pltpu.CompilerParams(use_tc_tiling_on_sc=False),
    scratch_types=[
        pltpu.VMEM((_NCH_PER_TILE, _CHUNK), jnp.int32),
        pltpu.VMEM((_NCH_PER_TILE, _CHUNK, _C), jnp.float32),
        pltpu.SemaphoreType.DMA,
        pltpu.SemaphoreType.DMA,
        pltpu.SemaphoreType.DMA,
    ],
)


def _invpack_body(i4_ref, bm_ref, o_ref):
    for d in range(2):
        o_ref[d] = jnp.dot(i4_ref[d], bm_ref[...],
                           preferred_element_type=jnp.float32)


def _layer_body(x_ref, hs_ref, inv_ref, f_ref, g_ref, b_ref, wb_ref, bias_ref,
                xo_ref):
    f = f_ref[...]
    acc = x_ref[...]
    for d in range(2):
        h = hs_ref[d] * inv_ref[d]
        m = jnp.mean(h, axis=0) @ f
        hc = h - m[None, :]
        v = jnp.mean(hc * hc, axis=0) @ f
        scale = g_ref[d] * lax.rsqrt(v + 1e-5)
        hn = hc * scale[None, :] + b_ref[d][None, :]
        o = jnp.dot(hn, wb_ref[d], preferred_element_type=jnp.float32)
        acc = acc + jnp.maximum(o + bias_ref[d][None, :], 0.0)
    xo_ref[...] = acc


def _final_body(x_ref, ff_ref, mw_ref, mb_ref, vw_ref, vb_ref, mean_ref, var_ref):
    xm = jnp.mean(x_ref[...], axis=0) @ ff_ref[...]
    mean_ref[...] = xm @ mw_ref[...].T + mb_ref[...]
    var_ref[...] = xm @ vw_ref[...].T + vb_ref[...]


_f32 = jnp.float32


def kernel(nodes, sources, targets, embed, bn_gamma, bn_beta, conv_W, conv_b,
           mean_W, mean_b, var_W, var_b):
    src = sources.astype(jnp.int32)
    tgt = targets.astype(jnp.int32)
    pad_g = jnp.zeros((_EP - _E,), jnp.int32)
    pad_s = jnp.full((_EP - _E,), _N, jnp.int32)
    g0 = jnp.concatenate([src, pad_g]).reshape(_EPC, _CHUNK)
    g1 = jnp.concatenate([tgt, pad_g]).reshape(_EPC, _CHUNK)
    s0 = jnp.concatenate([tgt, pad_s]).reshape(_EPC, _CHUNK)
    s1 = jnp.concatenate([src, pad_s]).reshape(_EPC, _CHUNK)
    gidx = jnp.stack([g0, g1])
    sidx = jnp.stack([s0, s1])
    zblk = jnp.zeros((_CHUNK, _C), _f32)
    nidx = jnp.concatenate(
        [nodes.astype(jnp.int32), jnp.zeros((_NODES_P - _N,), jnp.int32)]
    ).reshape(_NODES_P // _CHUNK, _CHUNK)

    npk = _N // 4
    fold = jnp.kron(jnp.ones((4, 4), _f32) / 4.0, jnp.eye(_C, dtype=_f32))
    foldf = jnp.kron(jnp.ones((4, 1), _f32) / 4.0, jnp.eye(_C, dtype=_f32))
    wb = jnp.kron(jnp.eye(4, dtype=_f32),
                  conv_W.transpose(0, 1, 3, 2))
    g4 = jnp.tile(bn_gamma, (1, 1, 4))
    b4 = jnp.tile(bn_beta, (1, 1, 4))
    bias4 = jnp.tile(conv_b, (1, 1, 4))

    izblk = jnp.zeros((_CR // _TILES, 128), _f32)
    cntp = _seg_cnt(sidx, izblk)
    invg = pl.pallas_call(
        _cntred_body, out_shape=jax.ShapeDtypeStruct((2, _CR, 128), _f32),
    )(cntp)
    inv4 = invg.reshape(2, _NPZ)[:, :_N].reshape(2, npk, 4)
    bmat = jnp.kron(jnp.eye(4, dtype=_f32), jnp.ones((1, _C), _f32))
    inv2 = pl.pallas_call(
        _invpack_body, out_shape=jax.ShapeDtypeStruct((2, npk, 128), _f32),
    )(inv4, bmat)
    x = _embed_gather(nidx, embed)[:_N].reshape(npk, 128)

    layer_call = pl.pallas_call(
        _layer_body, out_shape=jax.ShapeDtypeStruct((npk, 128), _f32),
    )
    for i in range(_L):
        hs2 = _seg_sum(gidx, sidx, x.reshape(_N, _C), zblk).reshape(2, npk, 128)
        x = layer_call(x, hs2, inv2, fold, g4[i], b4[i], wb[i], bias4[i])

    mean, var = pl.pallas_call(
        _final_body,
        out_shape=(jax.ShapeDtypeStruct((_D,), _f32),
                   jax.ShapeDtypeStruct((_D,), _f32)),
    )(x, foldf, mean_W, mean_b, var_W, var_b)
    return (mean, var)
